# Initial kernel scaffold; baseline (speedup 1.0000x reference)
import jax
import jax.numpy as jnp
from jax.experimental import pallas as pl


def kernel(x, up_index, up_attr, boundary_attr, boundary_index, W_msg_up, b_msg_up, W_up1, b_up1, W_up2, b_up2, W_b1, b_b1, W_b2, b_b2, W_comb, b_comb, eps1):
    raise NotImplementedError("write your pallas kernel here")


# speedup vs baseline: 2.6828x; 2.6828x over previous
import functools

import jax
import jax.numpy as jnp
from jax import lax
from jax.experimental import pallas as pl
from jax.experimental.pallas import tpu as pltpu
from jax.experimental.pallas import tpu_sc as plsc

NC, NS, LANES = 2, 16, 16
NW = NC * NS
CH = 128


def _mm_bias_body(a_ref, w_ref, b_ref, o_ref):
    o_ref[...] = (
        jnp.dot(a_ref[...], w_ref[...], preferred_element_type=jnp.float32)
        + b_ref[...]
    )


def _mm_body(a_ref, w_ref, o_ref):
    o_ref[...] = jnp.dot(a_ref[...], w_ref[...], preferred_element_type=jnp.float32)


def _tc_matmul_bias(a, w, b, bn):
    m, k = a.shape
    n = w.shape[1]
    return pl.pallas_call(
        _mm_bias_body,
        grid=(m // bn,),
        in_specs=[
            pl.BlockSpec((bn, k), lambda i: (i, 0)),
            pl.BlockSpec((k, n), lambda i: (0, 0)),
            pl.BlockSpec((1, n), lambda i: (0, 0)),
        ],
        out_specs=pl.BlockSpec((bn, n), lambda i: (i, 0)),
        out_shape=jax.ShapeDtypeStruct((m, n), jnp.float32),
    )(a, w, b)


def _tc_matmul(a, w, bn):
    m, k = a.shape
    n = w.shape[1]
    return pl.pallas_call(
        _mm_body,
        grid=(m // bn,),
        in_specs=[
            pl.BlockSpec((bn, k), lambda i: (i, 0)),
            pl.BlockSpec((k, n), lambda i: (0, 0)),
        ],
        out_specs=pl.BlockSpec((bn, n), lambda i: (i, 0)),
        out_shape=jax.ShapeDtypeStruct((m, n), jnp.float32),
    )(a, w)


def _final_body(x_ref, u0_ref, u1_ref, v0_ref, v1_ref,
                wu1_ref, bu1_ref, wu2_ref, bu2_ref,
                wb1_ref, bb1_ref, wb2_ref, bb2_ref,
                wc0_ref, wc1_ref, bc_ref, scale_ref, o_ref):
    scale = scale_ref[0, 0]
    xb = x_ref[...]
    h_up = u0_ref[...] + u1_ref[...] + scale * xb
    t = jnp.maximum(
        jnp.dot(h_up, wu1_ref[...], preferred_element_type=jnp.float32)
        + bu1_ref[...], 0.0)
    out_up = jnp.maximum(
        jnp.dot(t, wu2_ref[...], preferred_element_type=jnp.float32)
        + bu2_ref[...], 0.0)
    h_b = v0_ref[...] + v1_ref[...] + scale * xb
    t2 = jnp.maximum(
        jnp.dot(h_b, wb1_ref[...], preferred_element_type=jnp.float32)
        + bb1_ref[...], 0.0)
    out_b = jnp.maximum(
        jnp.dot(t2, wb2_ref[...], preferred_element_type=jnp.float32)
        + bb2_ref[...], 0.0)
    o_ref[...] = jnp.maximum(
        jnp.dot(out_up, wc0_ref[...], preferred_element_type=jnp.float32)
        + jnp.dot(out_b, wc1_ref[...], preferred_element_type=jnp.float32)
        + bc_ref[...], 0.0)


def _tc_final(x, u0, u1, v0, v1, wu1, bu1, wu2, bu2,
              wb1, bb1, wb2, bb2, wc0, wc1, bc, scale, bn):
    n_rows, d = x.shape
    mat = lambda: pl.BlockSpec((d, d), lambda i: (0, 0))
    vec = lambda: pl.BlockSpec((1, d), lambda i: (0, 0))
    rows = lambda: pl.BlockSpec((bn, d), lambda i: (i, 0))
    return pl.pallas_call(
        _final_body,
        grid=(n_rows // bn,),
        in_specs=[
            rows(), rows(), rows(), rows(), rows(),
            mat(), vec(), mat(), vec(),
            mat(), vec(), mat(), vec(),
            mat(), mat(), vec(),
            pl.BlockSpec(memory_space=pltpu.SMEM),
        ],
        out_specs=pl.BlockSpec((bn, d), lambda i: (i, 0)),
        out_shape=jax.ShapeDtypeStruct((n_rows, d), jnp.float32),
    )(x, u0, u1, v0, v1, wu1, bu1, wu2, bu2,
      wb1, bb1, wb2, bb2, wc0, wc1, bc, scale)


def _make_sc_agg(n_cells, d, e_edges, eb_pad, nt_rows, rps):
    epw = e_edges // NW
    full_e = epw // CH
    tail_e = epw - full_e * CH
    bpw = eb_pad // NW
    full_b = bpw // CH

    mesh = plsc.VectorSubcoreMesh(core_axis_name="c", subcore_axis_name="s")

    @functools.partial(
        pl.kernel,
        out_type=[jax.ShapeDtypeStruct((nt_rows, d), jnp.float32)] * 4,
        mesh=mesh,
        scratch_types=[
            pltpu.VMEM_SHARED((nt_rows, d), jnp.float32),
            pltpu.VMEM((CH, d), jnp.float32),
            pltpu.VMEM((CH, d), jnp.float32),
            pltpu.VMEM((CH,), jnp.int32),
            pltpu.VMEM((CH,), jnp.int32),
            pltpu.VMEM((tail_e, d), jnp.float32),
            pltpu.VMEM((tail_e, d), jnp.float32),
            pltpu.VMEM((tail_e,), jnp.int32),
            pltpu.VMEM((tail_e,), jnp.int32),
            pltpu.SemaphoreType.DMA,
        ],
    )
    def sc_agg(xp_hbm, ua_hbm, src_hbm, dst_hbm, bj_hbm, bi_hbm, battr_hbm,
               z_hbm, up0_hbm, up1_hbm, b0_hbm, b1_hbm,
               acc, ua_v, rows_v, si, di, ua_t, rows_t, si_t, di_t, sem):
        c = lax.axis_index("c")
        s = lax.axis_index("s")
        wid = s * NC + c
        row0 = s * rps

        def zero_acc():
            pltpu.sync_copy(z_hbm, acc.at[pl.ds(row0, rps)])

        def relu_add(rows, ua, nrows):
            def row_fn(r, carry):
                for cb in range(d // LANES):
                    sl = pl.ds(cb * LANES, LANES)
                    rows[r, sl] = jnp.maximum(rows[r, sl] + ua[r, sl], 0.0)
                return carry
            lax.fori_loop(0, nrows, row_fn, 0)

        zero_acc()
        plsc.subcore_barrier()

        ebase = wid * epw

        def edge_chunk(i, carry):
            off = ebase + i * CH
            pltpu.sync_copy(src_hbm.at[pl.ds(off, CH)], si)
            pltpu.sync_copy(dst_hbm.at[pl.ds(off, CH)], di)
            pltpu.sync_copy(ua_hbm.at[pl.ds(off, CH)], ua_v)
            pltpu.async_copy(xp_hbm.at[si], rows_v, sem).wait()
            relu_add(rows_v, ua_v, CH)
            pltpu.sync_copy(rows_v, acc.at[di], add=True)
            return carry

        lax.fori_loop(0, full_e, edge_chunk, 0)

        if tail_e:
            off = ebase + full_e * CH
            pltpu.sync_copy(src_hbm.at[pl.ds(off, tail_e)], si_t)
            pltpu.sync_copy(dst_hbm.at[pl.ds(off, tail_e)], di_t)
            pltpu.sync_copy(ua_hbm.at[pl.ds(off, tail_e)], ua_t)
            pltpu.async_copy(xp_hbm.at[si_t], rows_t, sem).wait()
            relu_add(rows_t, ua_t, tail_e)
            pltpu.sync_copy(rows_t, acc.at[di_t], add=True)

        plsc.subcore_barrier()

        @pl.when(c == 0)
        def _():
            pltpu.sync_copy(acc.at[pl.ds(row0, rps)], up0_hbm.at[pl.ds(row0, rps)])

        @pl.when(c == 1)
        def _():
            pltpu.sync_copy(acc.at[pl.ds(row0, rps)], up1_hbm.at[pl.ds(row0, rps)])

        zero_acc()
        plsc.subcore_barrier()

        bbase = wid * bpw

        def b_chunk(i, carry):
            off = bbase + i * CH
            pltpu.sync_copy(bj_hbm.at[pl.ds(off, CH)], si)
            pltpu.sync_copy(bi_hbm.at[pl.ds(off, CH)], di)
            pltpu.async_copy(battr_hbm.at[si], rows_v, sem).wait()
            pltpu.sync_copy(rows_v, acc.at[di], add=True)
            return carry

        lax.fori_loop(0, full_b, b_chunk, 0)
        plsc.subcore_barrier()

        @pl.when(c == 0)
        def _():
            pltpu.sync_copy(acc.at[pl.ds(row0, rps)], b0_hbm.at[pl.ds(row0, rps)])

        @pl.when(c == 1)
        def _():
            pltpu.sync_copy(acc.at[pl.ds(row0, rps)], b1_hbm.at[pl.ds(row0, rps)])

    return sc_agg


def kernel(x, up_index, up_attr, boundary_attr, boundary_index,
           W_msg_up, b_msg_up, W_up1, b_up1, W_up2, b_up2,
           W_b1, b_b1, W_b2, b_b2, W_comb, b_comb, eps1):
    n_cells, d = x.shape
    e_edges = up_attr.shape[0]
    eb = boundary_index.shape[1]

    rps = -(-(n_cells + 1) // NS)
    rps = -(-rps // 8) * 8
    nt_rows = rps * NS
    eb_pad = -(-eb // (NW * CH)) * (NW * CH)

    w_top = W_msg_up[:d]
    w_bot = W_msg_up[d:]

    ua = _tc_matmul_bias(up_attr, w_bot, b_msg_up.reshape(1, d), bn=2000)
    xp = _tc_matmul(x, w_top, bn=1000)

    src = up_index[0]
    dst = up_index[1]
    pad = eb_pad - eb
    bj = jnp.concatenate([boundary_index[0], jnp.zeros((pad,), jnp.int32)])
    bi = jnp.concatenate([boundary_index[1],
                          jnp.full((pad,), n_cells, jnp.int32)])
    zrows = jnp.zeros((rps, d), jnp.float32)

    sc_agg = _make_sc_agg(n_cells, d, e_edges, eb_pad, nt_rows, rps)
    up0, up1, b0, b1 = sc_agg(xp, ua, src, dst, bj, bi, boundary_attr, zrows)

    scale = (1.0 + eps1).reshape(1, 1)
    out = _tc_final(x, up0[:n_cells], up1[:n_cells], b0[:n_cells], b1[:n_cells],
                    W_up1, b_up1.reshape(1, d), W_up2, b_up2.reshape(1, d),
                    W_b1, b_b1.reshape(1, d), W_b2, b_b2.reshape(1, d),
                    W_comb[:d], W_comb[d:], b_comb.reshape(1, d),
                    scale, bn=1000)
    return out


# speedup vs baseline: 3.8015x; 1.4170x over previous
import functools

import jax
import jax.numpy as jnp
from jax import lax
from jax.experimental import pallas as pl
from jax.experimental.pallas import tpu as pltpu
from jax.experimental.pallas import tpu_sc as plsc

NC, NS, LANES = 2, 16, 16
NW = NC * NS
CH = 32
NBUF = 4


def _mm_bias_body(a_ref, w_ref, b_ref, o_ref):
    o_ref[...] = (
        jnp.dot(a_ref[...], w_ref[...], preferred_element_type=jnp.float32)
        + b_ref[...]
    )


def _mm_body(a_ref, w_ref, o_ref):
    o_ref[...] = jnp.dot(a_ref[...], w_ref[...], preferred_element_type=jnp.float32)


def _tc_matmul_bias(a, w, b, bn):
    m, k = a.shape
    n = w.shape[1]
    return pl.pallas_call(
        _mm_bias_body,
        grid=(m // bn,),
        in_specs=[
            pl.BlockSpec((bn, k), lambda i: (i, 0)),
            pl.BlockSpec((k, n), lambda i: (0, 0)),
            pl.BlockSpec((1, n), lambda i: (0, 0)),
        ],
        out_specs=pl.BlockSpec((bn, n), lambda i: (i, 0)),
        out_shape=jax.ShapeDtypeStruct((m, n), jnp.float32),
    )(a, w, b)


def _tc_matmul(a, w, bn):
    m, k = a.shape
    n = w.shape[1]
    return pl.pallas_call(
        _mm_body,
        grid=(m // bn,),
        in_specs=[
            pl.BlockSpec((bn, k), lambda i: (i, 0)),
            pl.BlockSpec((k, n), lambda i: (0, 0)),
        ],
        out_specs=pl.BlockSpec((bn, n), lambda i: (i, 0)),
        out_shape=jax.ShapeDtypeStruct((m, n), jnp.float32),
    )(a, w)


def _final_body(x_ref, u0_ref, u1_ref, v0_ref, v1_ref,
                wu1_ref, bu1_ref, wu2_ref, bu2_ref,
                wb1_ref, bb1_ref, wb2_ref, bb2_ref,
                wc0_ref, wc1_ref, bc_ref, scale_ref, o_ref):
    scale = scale_ref[0, 0]
    xb = x_ref[...]
    h_up = u0_ref[...] + u1_ref[...] + scale * xb
    t = jnp.maximum(
        jnp.dot(h_up, wu1_ref[...], preferred_element_type=jnp.float32)
        + bu1_ref[...], 0.0)
    out_up = jnp.maximum(
        jnp.dot(t, wu2_ref[...], preferred_element_type=jnp.float32)
        + bu2_ref[...], 0.0)
    h_b = v0_ref[...] + v1_ref[...] + scale * xb
    t2 = jnp.maximum(
        jnp.dot(h_b, wb1_ref[...], preferred_element_type=jnp.float32)
        + bb1_ref[...], 0.0)
    out_b = jnp.maximum(
        jnp.dot(t2, wb2_ref[...], preferred_element_type=jnp.float32)
        + bb2_ref[...], 0.0)
    o_ref[...] = jnp.maximum(
        jnp.dot(out_up, wc0_ref[...], preferred_element_type=jnp.float32)
        + jnp.dot(out_b, wc1_ref[...], preferred_element_type=jnp.float32)
        + bc_ref[...], 0.0)


def _tc_final(x, u0, u1, v0, v1, wu1, bu1, wu2, bu2,
              wb1, bb1, wb2, bb2, wc0, wc1, bc, scale, bn):
    n_rows, d = x.shape
    mat = lambda: pl.BlockSpec((d, d), lambda i: (0, 0))
    vec = lambda: pl.BlockSpec((1, d), lambda i: (0, 0))
    rows = lambda: pl.BlockSpec((bn, d), lambda i: (i, 0))
    return pl.pallas_call(
        _final_body,
        grid=(n_rows // bn,),
        in_specs=[
            rows(), rows(), rows(), rows(), rows(),
            mat(), vec(), mat(), vec(),
            mat(), vec(), mat(), vec(),
            mat(), mat(), vec(),
            pl.BlockSpec(memory_space=pltpu.SMEM),
        ],
        out_specs=pl.BlockSpec((bn, d), lambda i: (i, 0)),
        out_shape=jax.ShapeDtypeStruct((n_rows, d), jnp.float32),
    )(x, u0, u1, v0, v1, wu1, bu1, wu2, bu2,
      wb1, bb1, wb2, bb2, wc0, wc1, bc, scale)


def _make_sc_agg(n_cells, d, e_edges, eb_pad, nt_rows, rps):
    epw = e_edges // NW
    full_e = epw // CH
    tail_e = epw - full_e * CH
    bpw = eb_pad // NW
    full_b = bpw // CH
    assert full_e % NBUF == 0 and full_b % NBUF == 0

    mesh = plsc.VectorSubcoreMesh(core_axis_name="c", subcore_axis_name="s")

    @functools.partial(
        pl.kernel,
        out_type=[jax.ShapeDtypeStruct((nt_rows, d), jnp.float32)] * 4,
        mesh=mesh,
        scratch_types=[
            pltpu.VMEM_SHARED((nt_rows, d), jnp.float32),
            pltpu.VMEM((NBUF, CH, d), jnp.float32),
            pltpu.VMEM((NBUF, CH, d), jnp.float32),
            pltpu.VMEM((NBUF, CH), jnp.int32),
            pltpu.VMEM((NBUF, CH), jnp.int32),
            pltpu.VMEM((tail_e, d), jnp.float32),
            pltpu.VMEM((tail_e, d), jnp.float32),
            pltpu.VMEM((tail_e,), jnp.int32),
            pltpu.VMEM((tail_e,), jnp.int32),
            pltpu.SemaphoreType.DMA((NBUF,)),
            pltpu.SemaphoreType.DMA((NBUF,)),
            pltpu.SemaphoreType.DMA((NBUF,)),
            pltpu.SemaphoreType.DMA((NBUF,)),
            pltpu.SemaphoreType.DMA,
        ],
    )
    def sc_agg(xp_hbm, ua_hbm, src_hbm, dst_hbm, bj_hbm, bi_hbm, battr_hbm,
               z_hbm, up0_hbm, up1_hbm, b0_hbm, b1_hbm,
               acc, ua_v, rows_v, si, di, ua_t, rows_t, si_t, di_t,
               sem_i, sem_u, sem_g, sem_s, sem_t):
        c = lax.axis_index("c")
        s = lax.axis_index("s")
        wid = s * NC + c
        row0 = s * rps

        def zero_acc():
            pltpu.sync_copy(z_hbm, acc.at[pl.ds(row0, rps)])

        def relu_add(b):
            def row_fn(r2, carry):
                for dr in range(2):
                    r = r2 * 2 + dr
                    for cb in range(d // LANES):
                        sl = pl.ds(cb * LANES, LANES)
                        rows_v[b, r, sl] = jnp.maximum(
                            rows_v[b, r, sl] + ua_v[b, r, sl], 0.0)
                return carry
            lax.fori_loop(0, CH // 2, row_fn, 0)

        def pipeline(base, nfull, src_idx_hbm, dst_idx_hbm, tbl_hbm,
                     with_ua):
            def fire_idx(g, b):
                off = base + g * CH
                pltpu.async_copy(src_idx_hbm.at[pl.ds(off, CH)], si.at[b],
                                 sem_i.at[b])
                pltpu.async_copy(dst_idx_hbm.at[pl.ds(off, CH)], di.at[b],
                                 sem_i.at[b])
                if with_ua:
                    pltpu.async_copy(ua_hbm.at[pl.ds(off, CH)], ua_v.at[b],
                                     sem_u.at[b])

            def wait_idx(b):
                pltpu.make_async_copy(src_idx_hbm.at[pl.ds(0, CH)], si.at[b],
                                      sem_i.at[b]).wait()
                pltpu.make_async_copy(dst_idx_hbm.at[pl.ds(0, CH)], di.at[b],
                                      sem_i.at[b]).wait()

            def fire_gather(b):
                pltpu.async_copy(tbl_hbm.at[si.at[b]], rows_v.at[b],
                                 sem_g.at[b])

            def wait_gather(b):
                pltpu.make_async_copy(tbl_hbm.at[si.at[b]], rows_v.at[b],
                                      sem_g.at[b]).wait()

            def wait_ua(b):
                pltpu.make_async_copy(ua_hbm.at[pl.ds(0, CH)], ua_v.at[b],
                                      sem_u.at[b]).wait()

            def fire_scatter(b):
                pltpu.async_copy(rows_v.at[b], acc.at[di.at[b]], sem_s.at[b],
                                 add=True)

            def wait_scatter(b):
                pltpu.make_async_copy(rows_v.at[b], acc.at[di.at[b]],
                                      sem_s.at[b]).wait()

            fire_idx(0, 0)
            fire_idx(1, 1)
            wait_idx(0)
            fire_gather(0)

            def group(i, carry):
                g0 = i * NBUF
                for db in range(NBUF):
                    g = g0 + db
                    b2 = (db + 2) % NBUF

                    @pl.when(jnp.logical_and(g + 2 >= NBUF, g + 2 < nfull))
                    def _():
                        wait_scatter(b2)

                    @pl.when(g + 2 < nfull)
                    def _():
                        fire_idx(g + 2, b2)

                    b1 = (db + 1) % NBUF

                    @pl.when(g + 1 < nfull)
                    def _():
                        wait_idx(b1)
                        fire_gather(b1)

                    if with_ua:
                        wait_ua(db)
                    wait_gather(db)
                    if with_ua:
                        relu_add(db)
                    fire_scatter(db)
                return carry

            lax.fori_loop(0, nfull // NBUF, group, 0)
            for b in range(NBUF):
                wait_scatter(b)

        zero_acc()
        plsc.subcore_barrier()

        pipeline(wid * epw, full_e, src_hbm, dst_hbm, xp_hbm, with_ua=True)

        if tail_e:
            off = wid * epw + full_e * CH
            pltpu.sync_copy(src_hbm.at[pl.ds(off, tail_e)], si_t)
            pltpu.sync_copy(dst_hbm.at[pl.ds(off, tail_e)], di_t)
            pltpu.sync_copy(ua_hbm.at[pl.ds(off, tail_e)], ua_t)
            pltpu.async_copy(xp_hbm.at[si_t], rows_t, sem_t).wait()

            def trow(r, carry):
                for cb in range(d // LANES):
                    sl = pl.ds(cb * LANES, LANES)
                    rows_t[r, sl] = jnp.maximum(rows_t[r, sl] + ua_t[r, sl], 0.0)
                return carry
            lax.fori_loop(0, tail_e, trow, 0)
            pltpu.sync_copy(rows_t, acc.at[di_t], add=True)

        plsc.subcore_barrier()

        @pl.when(c == 0)
        def _():
            pltpu.sync_copy(acc.at[pl.ds(row0, rps)], up0_hbm.at[pl.ds(row0, rps)])

        @pl.when(c == 1)
        def _():
            pltpu.sync_copy(acc.at[pl.ds(row0, rps)], up1_hbm.at[pl.ds(row0, rps)])

        zero_acc()
        plsc.subcore_barrier()

        pipeline(wid * bpw, full_b, bj_hbm, bi_hbm, battr_hbm, with_ua=False)

        plsc.subcore_barrier()

        @pl.when(c == 0)
        def _():
            pltpu.sync_copy(acc.at[pl.ds(row0, rps)], b0_hbm.at[pl.ds(row0, rps)])

        @pl.when(c == 1)
        def _():
            pltpu.sync_copy(acc.at[pl.ds(row0, rps)], b1_hbm.at[pl.ds(row0, rps)])

    return sc_agg


def kernel(x, up_index, up_attr, boundary_attr, boundary_index,
           W_msg_up, b_msg_up, W_up1, b_up1, W_up2, b_up2,
           W_b1, b_b1, W_b2, b_b2, W_comb, b_comb, eps1):
    n_cells, d = x.shape
    e_edges = up_attr.shape[0]
    eb = boundary_index.shape[1]

    rps = -(-(n_cells + 1) // NS)
    rps = -(-rps // 8) * 8
    nt_rows = rps * NS
    eb_pad = -(-eb // (NW * CH * NBUF)) * (NW * CH * NBUF)

    w_top = W_msg_up[:d]
    w_bot = W_msg_up[d:]

    ua = _tc_matmul_bias(up_attr, w_bot, b_msg_up.reshape(1, d), bn=2000)
    xp = _tc_matmul(x, w_top, bn=1000)

    src = up_index[0]
    dst = up_index[1]
    pad = eb_pad - eb
    bj = jnp.concatenate([boundary_index[0], jnp.zeros((pad,), jnp.int32)])
    bi = jnp.concatenate([boundary_index[1],
                          jnp.full((pad,), n_cells, jnp.int32)])
    zrows = jnp.zeros((rps, d), jnp.float32)

    sc_agg = _make_sc_agg(n_cells, d, e_edges, eb_pad, nt_rows, rps)
    up0, up1, b0, b1 = sc_agg(xp, ua, src, dst, bj, bi, boundary_attr, zrows)

    scale = (1.0 + eps1).reshape(1, 1)
    out = _tc_final(x, up0[:n_cells], up1[:n_cells], b0[:n_cells], b1[:n_cells],
                    W_up1, b_up1.reshape(1, d), W_up2, b_up2.reshape(1, d),
                    W_b1, b_b1.reshape(1, d), W_b2, b_b2.reshape(1, d),
                    W_comb[:d], W_comb[d:], b_comb.reshape(1, d),
                    scale, bn=1000)
    return out


# speedup vs baseline: 3.8693x; 1.0179x over previous
import functools

import jax
import jax.numpy as jnp
from jax import lax
from jax.experimental import pallas as pl
from jax.experimental.pallas import tpu as pltpu
from jax.experimental.pallas import tpu_sc as plsc

NC, NS, LANES = 2, 16, 16
NW = NC * NS
CH = 32
NBUF = 4


def _mm_bias_body(a_ref, w_ref, b_ref, o_ref):
    o_ref[...] = (
        jnp.dot(a_ref[...], w_ref[...], preferred_element_type=jnp.float32)
        + b_ref[...]
    )


def _mm_body(a_ref, w_ref, o_ref):
    o_ref[...] = jnp.dot(a_ref[...], w_ref[...], preferred_element_type=jnp.float32)


def _tc_matmul_bias(a, w, b, bn):
    m, k = a.shape
    n = w.shape[1]
    return pl.pallas_call(
        _mm_bias_body,
        grid=(m // bn,),
        in_specs=[
            pl.BlockSpec((bn, k), lambda i: (i, 0)),
            pl.BlockSpec((k, n), lambda i: (0, 0)),
            pl.BlockSpec((1, n), lambda i: (0, 0)),
        ],
        out_specs=pl.BlockSpec((bn, n), lambda i: (i, 0)),
        out_shape=jax.ShapeDtypeStruct((m, n), jnp.float32),
    )(a, w, b)


def _tc_matmul(a, w, bn):
    m, k = a.shape
    n = w.shape[1]
    return pl.pallas_call(
        _mm_body,
        grid=(m // bn,),
        in_specs=[
            pl.BlockSpec((bn, k), lambda i: (i, 0)),
            pl.BlockSpec((k, n), lambda i: (0, 0)),
        ],
        out_specs=pl.BlockSpec((bn, n), lambda i: (i, 0)),
        out_shape=jax.ShapeDtypeStruct((m, n), jnp.float32),
    )(a, w)


def _final_body(x_ref, u0_ref, u1_ref, v0_ref, v1_ref,
                wu1_ref, bu1_ref, wu2_ref, bu2_ref,
                wb1_ref, bb1_ref, wb2_ref, bb2_ref,
                wc0_ref, wc1_ref, bc_ref, scale_ref, o_ref):
    scale = scale_ref[0, 0]
    xb = x_ref[...]
    h_up = u0_ref[...] + u1_ref[...] + scale * xb
    t = jnp.maximum(
        jnp.dot(h_up, wu1_ref[...], preferred_element_type=jnp.float32)
        + bu1_ref[...], 0.0)
    out_up = jnp.maximum(
        jnp.dot(t, wu2_ref[...], preferred_element_type=jnp.float32)
        + bu2_ref[...], 0.0)
    h_b = v0_ref[...] + v1_ref[...] + scale * xb
    t2 = jnp.maximum(
        jnp.dot(h_b, wb1_ref[...], preferred_element_type=jnp.float32)
        + bb1_ref[...], 0.0)
    out_b = jnp.maximum(
        jnp.dot(t2, wb2_ref[...], preferred_element_type=jnp.float32)
        + bb2_ref[...], 0.0)
    o_ref[...] = jnp.maximum(
        jnp.dot(out_up, wc0_ref[...], preferred_element_type=jnp.float32)
        + jnp.dot(out_b, wc1_ref[...], preferred_element_type=jnp.float32)
        + bc_ref[...], 0.0)


def _tc_final(x, u0, u1, v0, v1, wu1, bu1, wu2, bu2,
              wb1, bb1, wb2, bb2, wc0, wc1, bc, scale, bn):
    n_rows, d = x.shape
    mat = lambda: pl.BlockSpec((d, d), lambda i: (0, 0))
    vec = lambda: pl.BlockSpec((1, d), lambda i: (0, 0))
    rows = lambda: pl.BlockSpec((bn, d), lambda i: (i, 0))
    return pl.pallas_call(
        _final_body,
        grid=(n_rows // bn,),
        in_specs=[
            rows(), rows(), rows(), rows(), rows(),
            mat(), vec(), mat(), vec(),
            mat(), vec(), mat(), vec(),
            mat(), mat(), vec(),
            pl.BlockSpec(memory_space=pltpu.SMEM),
        ],
        out_specs=pl.BlockSpec((bn, d), lambda i: (i, 0)),
        out_shape=jax.ShapeDtypeStruct((n_rows, d), jnp.float32),
    )(x, u0, u1, v0, v1, wu1, bu1, wu2, bu2,
      wb1, bb1, wb2, bb2, wc0, wc1, bc, scale)


def _make_sc_pass(d, n_pairs, nt_rows, rps, with_ua):
    ppw = n_pairs // NW
    full = ppw // CH
    tail_n = ppw - full * CH
    assert full % NBUF == 0

    mesh = plsc.VectorSubcoreMesh(core_axis_name="c", subcore_axis_name="s")

    scratch = [
        pltpu.VMEM_SHARED((nt_rows, d), jnp.float32),
        pltpu.VMEM((NBUF, CH, d), jnp.float32),
        pltpu.VMEM((NBUF, CH), jnp.int32),
        pltpu.VMEM((NBUF, CH), jnp.int32),
        pltpu.SemaphoreType.DMA((NBUF,)),
        pltpu.SemaphoreType.DMA((NBUF,)),
        pltpu.SemaphoreType.DMA((NBUF,)),
        pltpu.SemaphoreType.DMA,
    ]
    if with_ua:
        scratch += [
            pltpu.VMEM((NBUF, CH, d), jnp.float32),
            pltpu.SemaphoreType.DMA((NBUF,)),
        ]
    if tail_n:
        scratch += [
            pltpu.VMEM((tail_n, d), jnp.float32),
            pltpu.VMEM((tail_n,), jnp.int32),
            pltpu.VMEM((tail_n,), jnp.int32),
        ]
        if with_ua:
            scratch += [pltpu.VMEM((tail_n, d), jnp.float32)]

    @functools.partial(
        pl.kernel,
        out_type=[jax.ShapeDtypeStruct((nt_rows, d), jnp.float32)] * 2,
        mesh=mesh,
        scratch_types=scratch,
    )
    def sc_pass(*refs):
        it = iter(refs)
        tbl_hbm = next(it)
        ua_hbm = next(it) if with_ua else None
        src_hbm = next(it)
        dst_hbm = next(it)
        z_hbm = next(it)
        o0_hbm = next(it)
        o1_hbm = next(it)
        acc = next(it)
        rows_v = next(it)
        si = next(it)
        di = next(it)
        sem_i = next(it)
        sem_g = next(it)
        sem_s = next(it)
        sem_t = next(it)
        if with_ua:
            ua_v = next(it)
            sem_u = next(it)
        if tail_n:
            rows_t = next(it)
            si_t = next(it)
            di_t = next(it)
            if with_ua:
                ua_t = next(it)

        c = lax.axis_index("c")
        s = lax.axis_index("s")
        wid = s * NC + c
        row0 = s * rps
        base = wid * ppw
        nfull = full

        def relu_add(b):
            def row_fn(r2, carry):
                for dr in range(2):
                    r = r2 * 2 + dr
                    for cb in range(d // LANES):
                        sl = pl.ds(cb * LANES, LANES)
                        rows_v[b, r, sl] = jnp.maximum(
                            rows_v[b, r, sl] + ua_v[b, r, sl], 0.0)
                return carry
            lax.fori_loop(0, CH // 2, row_fn, 0)

        def pipeline():
            def fire_idx(g, b):
                off = base + g * CH
                pltpu.async_copy(src_hbm.at[pl.ds(off, CH)], si.at[b],
                                 sem_i.at[b])
                pltpu.async_copy(dst_hbm.at[pl.ds(off, CH)], di.at[b],
                                 sem_i.at[b])
                if with_ua:
                    pltpu.async_copy(ua_hbm.at[pl.ds(off, CH)], ua_v.at[b],
                                     sem_u.at[b])

            def wait_idx(b):
                pltpu.make_async_copy(src_hbm.at[pl.ds(0, CH)], si.at[b],
                                      sem_i.at[b]).wait()
                pltpu.make_async_copy(dst_hbm.at[pl.ds(0, CH)], di.at[b],
                                      sem_i.at[b]).wait()

            def fire_gather(b):
                pltpu.async_copy(tbl_hbm.at[si.at[b]], rows_v.at[b],
                                 sem_g.at[b])

            def wait_gather(b):
                pltpu.make_async_copy(tbl_hbm.at[si.at[b]], rows_v.at[b],
                                      sem_g.at[b]).wait()

            def wait_ua(b):
                pltpu.make_async_copy(ua_hbm.at[pl.ds(0, CH)], ua_v.at[b],
                                      sem_u.at[b]).wait()

            def fire_scatter(b):
                pltpu.async_copy(rows_v.at[b], acc.at[di.at[b]], sem_s.at[b],
                                 add=True)

            def wait_scatter(b):
                pltpu.make_async_copy(rows_v.at[b], acc.at[di.at[b]],
                                      sem_s.at[b]).wait()

            fire_idx(0, 0)
            fire_idx(1, 1)
            wait_idx(0)
            fire_gather(0)

            def group(i, carry):
                g0 = i * NBUF
                for db in range(NBUF):
                    g = g0 + db
                    b2 = (db + 2) % NBUF

                    @pl.when(jnp.logical_and(g + 2 >= NBUF, g + 2 < nfull))
                    def _():
                        wait_scatter(b2)

                    @pl.when(g + 2 < nfull)
                    def _():
                        fire_idx(g + 2, b2)

                    b1 = (db + 1) % NBUF

                    @pl.when(g + 1 < nfull)
                    def _():
                        wait_idx(b1)
                        fire_gather(b1)

                    if with_ua:
                        wait_ua(db)
                    wait_gather(db)
                    if with_ua:
                        relu_add(db)
                    fire_scatter(db)
                return carry

            lax.fori_loop(0, nfull // NBUF, group, 0)
            for b in range(NBUF):
                wait_scatter(b)

        pltpu.sync_copy(z_hbm, acc.at[pl.ds(row0, rps)])
        plsc.subcore_barrier()

        pipeline()

        if tail_n:
            off = base + full * CH
            pltpu.sync_copy(src_hbm.at[pl.ds(off, tail_n)], si_t)
            pltpu.sync_copy(dst_hbm.at[pl.ds(off, tail_n)], di_t)
            if with_ua:
                pltpu.sync_copy(ua_hbm.at[pl.ds(off, tail_n)], ua_t)
            pltpu.async_copy(tbl_hbm.at[si_t], rows_t, sem_t).wait()

            if with_ua:
                def trow(r, carry):
                    for cb in range(d // LANES):
                        sl = pl.ds(cb * LANES, LANES)
                        rows_t[r, sl] = jnp.maximum(
                            rows_t[r, sl] + ua_t[r, sl], 0.0)
                    return carry
                lax.fori_loop(0, tail_n, trow, 0)
            pltpu.sync_copy(rows_t, acc.at[di_t], add=True)

        plsc.subcore_barrier()

        @pl.when(c == 0)
        def _():
            pltpu.sync_copy(acc.at[pl.ds(row0, rps)], o0_hbm.at[pl.ds(row0, rps)])

        @pl.when(c == 1)
        def _():
            pltpu.sync_copy(acc.at[pl.ds(row0, rps)], o1_hbm.at[pl.ds(row0, rps)])

    return sc_pass


def kernel(x, up_index, up_attr, boundary_attr, boundary_index,
           W_msg_up, b_msg_up, W_up1, b_up1, W_up2, b_up2,
           W_b1, b_b1, W_b2, b_b2, W_comb, b_comb, eps1):
    n_cells, d = x.shape
    e_edges = up_attr.shape[0]
    eb = boundary_index.shape[1]

    rps = -(-(n_cells + 1) // NS)
    rps = -(-rps // 8) * 8
    nt_rows = rps * NS
    eb_pad = -(-eb // (NW * CH * NBUF)) * (NW * CH * NBUF)

    w_top = W_msg_up[:d]
    w_bot = W_msg_up[d:]

    src = up_index[0]
    dst = up_index[1]
    pad = eb_pad - eb
    bj = jnp.concatenate([boundary_index[0], jnp.zeros((pad,), jnp.int32)])
    bi = jnp.concatenate([boundary_index[1],
                          jnp.full((pad,), n_cells, jnp.int32)])
    zrows = jnp.zeros((rps, d), jnp.float32)

    sc_bnd = _make_sc_pass(d, eb_pad, nt_rows, rps, with_ua=False)
    b0, b1 = sc_bnd(boundary_attr, bj, bi, zrows)

    ua = _tc_matmul_bias(up_attr, w_bot, b_msg_up.reshape(1, d), bn=2000)
    xp = _tc_matmul(x, w_top, bn=1000)

    sc_edge = _make_sc_pass(d, e_edges, nt_rows, rps, with_ua=True)
    up0, up1 = sc_edge(xp, ua, src, dst, zrows)

    scale = (1.0 + eps1).reshape(1, 1)
    out = _tc_final(x, up0, up1, b0, b1,
                    W_up1, b_up1.reshape(1, d), W_up2, b_up2.reshape(1, d),
                    W_b1, b_b1.reshape(1, d), W_b2, b_b2.reshape(1, d),
                    W_comb[:d], W_comb[d:], b_comb.reshape(1, d),
                    scale, bn=1000)
    return out
